# baseline (speedup 1.0000x reference)
import jax
import jax.numpy as jnp
from jax.experimental import pallas as pl
from jax.experimental.pallas import tpu as pltpu

_B, _C, _W, _H = 32, 19, 128, 128
_PIX = _W * _H


_IB = 8


def _body(x_ref, o_ref):
    for i in range(_IB):
        data = x_ref[i]
        t = jnp.swapaxes(data, 1, 2)
        t2 = t.reshape(_C, _PIX)
        s = jnp.sum(t2, axis=0, keepdims=True)
        o_ref[:, i * _PIX:(i + 1) * _PIX] = t2 * (1.0 / s)


def kernel(x):
    y = pl.pallas_call(
        _body,
        grid=(_B // _IB,),
        in_specs=[pl.BlockSpec((_IB, _C, _W, _H), lambda b: (b, 0, 0, 0))],
        out_specs=pl.BlockSpec((_C, _IB * _PIX), lambda b: (0, b)),
        out_shape=jax.ShapeDtypeStruct((_C, _B * _PIX), jnp.float32),
        compiler_params=pltpu.CompilerParams(
            dimension_semantics=("arbitrary",),
        ),
    )(x)
    return y.T
